# Initial kernel scaffold; baseline (speedup 1.0000x reference)
import jax
import jax.numpy as jnp
from jax.experimental import pallas as pl


def kernel(hidden_states, gate_w, w1, w3, w2):
    raise NotImplementedError("write your pallas kernel here")


# speedup vs baseline: 1.1352x; 1.1352x over previous
import functools

import jax
import jax.numpy as jnp
from jax import lax
from jax.experimental import pallas as pl
from jax.experimental.pallas import tpu as pltpu
from jax.experimental.pallas import tpu_sc as plsc

T, H, F, E, K = 2048, 1024, 3584, 8, 2
N = T * K
BM = 256
NT = N // BM + E - 1
NPAD = NT * BM
BF = 512
NF = F // BF

L = 16
NSUB = 16
NCORE = 2
HH = H // NCORE


def _router_body(x_ref, g_ref, i1_ref, i2_ref, p1_ref, p2_ref):
    x = x_ref[...]
    g = g_ref[...]
    logits = lax.dot_general(x, g, (((1,), (1,)), ((), ())),
                             preferred_element_type=jnp.float32)
    neg = jnp.float32(-jnp.inf)
    m1 = jnp.full((T, 1), neg)
    m2 = jnp.full((T, 1), neg)
    i1 = jnp.zeros((T, 1), jnp.int32)
    i2 = jnp.zeros((T, 1), jnp.int32)
    for e in range(E):
        le = logits[:, e:e + 1]
        upd1 = le > m1
        upd2 = jnp.logical_and(jnp.logical_not(upd1), le > m2)
        m2 = jnp.where(upd1, m1, jnp.where(upd2, le, m2))
        i2 = jnp.where(upd1, i1, jnp.where(upd2, e, i2))
        m1 = jnp.where(upd1, le, m1)
        i1 = jnp.where(upd1, e, i1)
    p1 = 1.0 / (1.0 + jnp.exp(m2 - m1))
    i1_ref[...] = i1
    i2_ref[...] = i2
    p1_ref[...] = p1
    p2_ref[...] = 1.0 - p1


def _router(x, gate_w):
    return pl.pallas_call(
        _router_body,
        out_shape=(jax.ShapeDtypeStruct((T, 1), jnp.int32),
                   jax.ShapeDtypeStruct((T, 1), jnp.int32),
                   jax.ShapeDtypeStruct((T, 1), jnp.float32),
                   jax.ShapeDtypeStruct((T, 1), jnp.float32)),
    )(x, gate_w)


PPW = N // NSUB
VPW = PPW // L
CPW = NPAD // NSUB

_sc_mesh = plsc.VectorSubcoreMesh(core_axis_name="c", subcore_axis_name="s")


def _dispatch_body(e_hbm, p_hbm, rt_hbm, rp_hbm, te_hbm, pos_hbm,
                   e_all, p_v, te_v, pos_a, pos_b, tok_v,
                   zb_i, zb_f):
    c = lax.axis_index("c")
    w = lax.axis_index("s")
    lane = lax.iota(jnp.int32, L)

    @pl.when(c == 0)
    def _active():
        pltpu.sync_copy(e_hbm, e_all)
        pltpu.sync_copy(p_hbm.at[pl.ds(w * PPW, PPW)], p_v)
        zi = jnp.zeros((L,), jnp.int32)
        zf = jnp.zeros((L,), jnp.float32)
        for j in range(CPW // L):
            zb_i[pl.ds(j * L, L)] = zi
            zb_f[pl.ds(j * L, L)] = zf
        pltpu.sync_copy(zb_i, rt_hbm.at[pl.ds(w * CPW, CPW)])
        pltpu.sync_copy(zb_f, rp_hbm.at[pl.ds(w * CPW, CPW)])

    plsc.subcore_barrier()

    @pl.when(c == 0)
    def _active2():
        myvec0 = w * VPW

        def _scan(i, carry):
            tot, pre = carry
            ev = e_all[pl.ds(i * L, L)]
            msk = (i < myvec0).astype(jnp.int32)
            for v in range(E):
                pc = plsc.all_reduce_population_count(ev == v)
                onev = jnp.where(lane == v, pc, 0)
                tot = tot + onev
                pre = pre + onev * msk
            return tot, pre

        totals, prefix = lax.fori_loop(
            0, N // L, _scan,
            (jnp.zeros((L,), jnp.int32), jnp.zeros((L,), jnp.int32)))
        pad = (totals + (BM - 1)) & (-BM)
        ends = plsc.cumsum(pad)
        starts_g = ends - pad
        cursor = starts_g + prefix

        @pl.when(w == 0)
        def _tiles():
            for half in range(2):
                ts = (lane + half * L) * BM
                te = jnp.zeros((L,), jnp.int32)
                for v in range(E):
                    end_v = jnp.sum(jnp.where(lane == v, ends, 0))
                    te = te + (ts >= end_v).astype(jnp.int32)
                te = jnp.minimum(te, E - 1)
                te_v[pl.ds(half * L, L)] = te
            pltpu.sync_copy(te_v, te_hbm)

        for j in range(VPW):
            ev = e_all[pl.ds(w * PPW + j * L, L)]
            base = jnp.zeros((L,), jnp.int32)
            rank = jnp.zeros((L,), jnp.int32)
            for v in range(E):
                m = ev == v
                cur_v = jnp.sum(jnp.where(lane == v, cursor, 0))
                base = jnp.where(m, cur_v, base)
                cs = plsc.cumsum(m.astype(jnp.int32))
                rank = jnp.where(m, cs - 1, rank)
                pc = plsc.all_reduce_population_count(m)
                cursor = cursor + jnp.where(lane == v, pc, 0)
            pos = base + rank
            tok = lax.shift_right_logical(w * PPW + j * L + lane, 1)
            if j < VPW // 2:
                pos_a[pl.ds(j * L, L)] = pos
            else:
                pos_b[pl.ds((j - VPW // 2) * L, L)] = pos
            tok_v[pl.ds(j * L, L)] = tok

        half_n = PPW // 2
        pltpu.sync_copy(tok_v.at[pl.ds(0, half_n)], rt_hbm.at[pos_a])
        pltpu.sync_copy(tok_v.at[pl.ds(half_n, half_n)], rt_hbm.at[pos_b])
        pltpu.sync_copy(p_v.at[pl.ds(0, half_n)], rp_hbm.at[pos_a])
        pltpu.sync_copy(p_v.at[pl.ds(half_n, half_n)], rp_hbm.at[pos_b])
        pltpu.sync_copy(pos_a, pos_hbm.at[pl.ds(w * PPW, half_n)])
        pltpu.sync_copy(pos_b, pos_hbm.at[pl.ds(w * PPW + half_n, half_n)])


def _dispatch(e_flat, p_flat):
    return pl.kernel(
        _dispatch_body,
        out_type=(jax.ShapeDtypeStruct((NPAD,), jnp.int32),
                  jax.ShapeDtypeStruct((NPAD,), jnp.float32),
                  jax.ShapeDtypeStruct((2 * L,), jnp.int32),
                  jax.ShapeDtypeStruct((N,), jnp.int32)),
        mesh=_sc_mesh,
        compiler_params=pltpu.CompilerParams(needs_layout_passes=False),
        scratch_types=[
            pltpu.VMEM((N,), jnp.int32),
            pltpu.VMEM((PPW,), jnp.float32),
            pltpu.VMEM((2 * L,), jnp.int32),
            pltpu.VMEM((PPW // 2,), jnp.int32),
            pltpu.VMEM((PPW // 2,), jnp.int32),
            pltpu.VMEM((PPW,), jnp.int32),
            pltpu.VMEM((CPW,), jnp.int32),
            pltpu.VMEM((CPW,), jnp.float32),
        ],
    )(e_flat, p_flat)


RPW = NPAD // (NSUB * NCORE)
GC0, GC1 = 96, 88


def _gather_body(rt_hbm, x_hbm, xs_hbm, idx_a, idx_b, buf, sem):
    wid = lax.axis_index("s") * NCORE + lax.axis_index("c")
    base = wid * RPW
    pltpu.sync_copy(rt_hbm.at[pl.ds(base, GC0)], idx_a)
    pltpu.sync_copy(rt_hbm.at[pl.ds(base + GC0, GC1)], idx_b)
    pltpu.async_copy(x_hbm.at[idx_a], buf, sem).wait()
    pltpu.sync_copy(buf, xs_hbm.at[pl.ds(base, GC0)])
    pltpu.async_copy(x_hbm.at[idx_b], buf.at[pl.ds(0, GC1)], sem).wait()
    pltpu.sync_copy(buf.at[pl.ds(0, GC1)], xs_hbm.at[pl.ds(base + GC0, GC1)])


def _gather(row_token, hidden):
    return pl.kernel(
        _gather_body,
        out_type=jax.ShapeDtypeStruct((NPAD, H), jnp.float32),
        mesh=_sc_mesh,
        scratch_types=[
            pltpu.VMEM((GC0,), jnp.int32),
            pltpu.VMEM((GC1,), jnp.int32),
            pltpu.VMEM((GC0, H), jnp.float32),
            pltpu.SemaphoreType.DMA,
        ],
    )(row_token, hidden)


def _swiglu_body(e_ref, x_ref, w1_ref, w3_ref, w2_ref, prob_ref, out_ref):
    f = pl.program_id(1)

    @pl.when(f == 0)
    def _():
        out_ref[...] = jnp.zeros_like(out_ref)

    x = x_ref[...]
    w1 = w1_ref[0]
    w3 = w3_ref[0]
    w2 = w2_ref[0]
    a = lax.dot_general(x, w1, (((1,), (1,)), ((), ())),
                        preferred_element_type=jnp.float32)
    b = lax.dot_general(x, w3, (((1,), (1,)), ((), ())),
                        preferred_element_type=jnp.float32)
    h = (a * lax.logistic(a)) * b
    out_ref[...] += lax.dot_general(h, w2, (((1,), (1,)), ((), ())),
                                    preferred_element_type=jnp.float32)

    @pl.when(f == NF - 1)
    def _():
        out_ref[...] *= prob_ref[...]


def _grouped_swiglu(x_sorted, w1, w3, w2, row_prob, tile_expert):
    grid_spec = pltpu.PrefetchScalarGridSpec(
        num_scalar_prefetch=1,
        grid=(NT, NF),
        in_specs=[
            pl.BlockSpec((BM, H), lambda t, f, e_ref: (t, 0)),
            pl.BlockSpec((1, BF, H), lambda t, f, e_ref: (e_ref[t], f, 0)),
            pl.BlockSpec((1, BF, H), lambda t, f, e_ref: (e_ref[t], f, 0)),
            pl.BlockSpec((1, H, BF), lambda t, f, e_ref: (e_ref[t], 0, f)),
            pl.BlockSpec((BM, 1), lambda t, f, e_ref: (t, 0)),
        ],
        out_specs=pl.BlockSpec((BM, H), lambda t, f, e_ref: (t, 0)),
    )
    return pl.pallas_call(
        _swiglu_body,
        grid_spec=grid_spec,
        out_shape=jax.ShapeDtypeStruct((NPAD, H), jnp.float32),
    )(tile_expert, x_sorted, w1, w3, w2, row_prob.reshape(NPAD, 1))


TPW = T // (NSUB * NCORE)
TCH = 32
HVEC = H // L


def _combine_body(y_hbm, p0_hbm, p1_hbm, out_hbm,
                  idx0, idx1, buf0, buf1, sem0, sem1):
    wid = lax.axis_index("s") * NCORE + lax.axis_index("c")
    for ch in range(TPW // TCH):
        t0 = wid * TPW + ch * TCH
        pltpu.sync_copy(p0_hbm.at[pl.ds(t0, TCH)], idx0)
        pltpu.sync_copy(p1_hbm.at[pl.ds(t0, TCH)], idx1)
        cp0 = pltpu.async_copy(y_hbm.at[idx0], buf0, sem0)
        cp1 = pltpu.async_copy(y_hbm.at[idx1], buf1, sem1)
        cp0.wait()
        cp1.wait()

        def _row(r, carry):
            for u in range(HVEC):
                col = u * L
                buf0[r, pl.ds(col, L)] = (buf0[r, pl.ds(col, L)]
                                          + buf1[r, pl.ds(col, L)])
            return carry

        lax.fori_loop(0, TCH, _row, 0)
        pltpu.sync_copy(buf0, out_hbm.at[pl.ds(t0, TCH)])


def _combine(y_sorted, pos0, pos1):
    return pl.kernel(
        _combine_body,
        out_type=jax.ShapeDtypeStruct((T, H), jnp.float32),
        mesh=_sc_mesh,
        scratch_types=[
            pltpu.VMEM((TCH,), jnp.int32),
            pltpu.VMEM((TCH,), jnp.int32),
            pltpu.VMEM((TCH, H), jnp.float32),
            pltpu.VMEM((TCH, H), jnp.float32),
            pltpu.SemaphoreType.DMA,
            pltpu.SemaphoreType.DMA,
        ],
    )(y_sorted, pos0, pos1)


def kernel(hidden_states, gate_w, w1, w3, w2):
    i1, i2, p1, p2 = _router(hidden_states, gate_w)
    e_flat = jnp.concatenate([i1, i2], axis=1).reshape(N)
    p_flat = jnp.concatenate([p1, p2], axis=1).reshape(N)
    row_token, row_prob, tile_expert, pos = _dispatch(e_flat, p_flat)
    x_sorted = _gather(row_token, hidden_states)
    y = _grouped_swiglu(x_sorted, w1, w3, w2, row_prob, tile_expert)
    pos2 = pos.reshape(T, K)
    return _combine(y, pos2[:, 0], pos2[:, 1])
